# Initial kernel scaffold; baseline (speedup 1.0000x reference)
import jax
import jax.numpy as jnp
from jax.experimental import pallas as pl


def kernel(curr_coords, curr_feat, skip_coords, skip_feat, upsampling_idxs, W_proj, b_proj, g_proj, be_proj, W_skip, b_skip, g_skip, be_skip):
    raise NotImplementedError("write your pallas kernel here")


# speedup vs baseline: 14.2920x; 14.2920x over previous
import functools

import jax
import jax.numpy as jnp
from jax import lax
from jax.experimental import pallas as pl
from jax.experimental.pallas import tpu as pltpu
from jax.experimental.pallas import tpu_sc as plsc

_B, _M, _N, _K = 8, 4096, 16384, 3
_CF, _CS, _CO = 64, 32, 64
_BN = _B * _N
_EPS = 1e-5


_NW = 32
_PTS = _BN // _NW
_CH = 128
_NCH = _PTS // _CH


def _gather_max_body(idx_hbm, table_hbm, out_hbm, fidx_v, rows_v,
                     out_v, sem):
    wid = lax.axis_index("s") * 2 + lax.axis_index("c")
    base = wid * _PTS
    row_off = (base // _N) * _M

    def chunk(j, carry):
        p0 = base + j * _CH
        for k in range(_K):
            pltpu.sync_copy(idx_hbm.at[pl.ds(k * _BN + p0, _CH)],
                            fidx_v.at[k])
        for k in range(_K):
            for g in range(_CH // 16):
                sl = pl.ds(g * 16, 16)
                fidx_v[k, sl] = fidx_v[k, sl] + row_off
        cps = [
            pltpu.async_copy(table_hbm.at[fidx_v.at[k]], rows_v.at[k], sem)
            for k in range(_K)
        ]
        for cp in cps:
            cp.wait()

        def point(p, c2):
            for c in range(_CF // 16):
                sl = pl.ds(c * 16, 16)
                m = jnp.maximum(rows_v[0, p, sl], rows_v[1, p, sl])
                out_v[p, sl] = jnp.maximum(m, rows_v[2, p, sl])
            return c2

        lax.fori_loop(0, _CH, point, 0)
        pltpu.sync_copy(out_v, out_hbm.at[pl.ds(p0, _CH)])
        return carry

    lax.fori_loop(0, _NCH, chunk, 0)


@functools.cache
def _gather_max_kernel():
    return pl.kernel(
        _gather_max_body,
        out_type=jax.ShapeDtypeStruct((_BN, _CF), jnp.float32),
        mesh=plsc.VectorSubcoreMesh(core_axis_name="c", subcore_axis_name="s"),
        compiler_params=pltpu.CompilerParams(use_tc_tiling_on_sc=False),
        scratch_types=[
            pltpu.VMEM((_K, _CH), jnp.int32),
            pltpu.VMEM((_K, _CH, _CF), jnp.float32),
            pltpu.VMEM((_CH, _CF), jnp.float32),
            pltpu.SemaphoreType.DMA,
        ],
    )


def _gather_max(idx_flat, table):
    return _gather_max_kernel()(idx_flat, table)


_RB_S = 2048


def _stats_body(x_ref, s_ref, wp_ref, bp_ref, ws_ref, bs_ref, o_ref):
    i = pl.program_id(0)

    @pl.when(i == 0)
    def _init():
        o_ref[...] = jnp.zeros_like(o_ref)

    y = jnp.dot(x_ref[...], wp_ref[...],
                preferred_element_type=jnp.float32) + bp_ref[...]
    s = jnp.dot(s_ref[...], ws_ref[...],
                preferred_element_type=jnp.float32) + bs_ref[...]
    acc = jnp.concatenate(
        [
            jnp.sum(y, 0, keepdims=True),
            jnp.sum(y * y, 0, keepdims=True),
            jnp.sum(s, 0, keepdims=True),
            jnp.sum(s * s, 0, keepdims=True),
            jnp.zeros((4, _CO), jnp.float32),
        ],
        axis=0,
    )
    o_ref[...] += acc


def _stats_call(x, s, wp, bp, ws, bs):
    nblk = _BN // _RB_S
    return pl.pallas_call(
        _stats_body,
        grid=(nblk,),
        in_specs=[
            pl.BlockSpec((_RB_S, _CF), lambda i: (i, 0)),
            pl.BlockSpec((_RB_S, _CS), lambda i: (i, 0)),
            pl.BlockSpec((_CF, _CO), lambda i: (0, 0)),
            pl.BlockSpec((1, _CO), lambda i: (0, 0)),
            pl.BlockSpec((_CS, _CO), lambda i: (0, 0)),
            pl.BlockSpec((1, _CO), lambda i: (0, 0)),
        ],
        out_specs=pl.BlockSpec((8, _CO), lambda i: (0, 0)),
        out_shape=jax.ShapeDtypeStruct((8, _CO), jnp.float32),
    )(x, s, wp, bp, ws, bs)


_RB_F = 1024


def _final_body(x_ref, s_ref, c_ref, st_ref, wp_ref, bp_ref, gp_ref, bep_ref,
                ws_ref, bs_ref, gs_ref, bes_ref, o_ref):
    nb = jnp.float32(_BN)
    st = st_ref[...]
    m_y = st[0:1] / nb
    v_y = st[1:2] / nb - m_y * m_y
    sc_y = gp_ref[...] * lax.rsqrt(v_y + _EPS)
    sh_y = bep_ref[...] - m_y * sc_y
    m_s = st[2:3] / nb
    v_s = st[3:4] / nb - m_s * m_s
    sc_s = gs_ref[...] * lax.rsqrt(v_s + _EPS)
    sh_s = bes_ref[...] - m_s * sc_s

    x = jnp.dot(x_ref[...], wp_ref[...] * sc_y,
                preferred_element_type=jnp.float32)
    x = jnp.maximum(x + (bp_ref[...] * sc_y + sh_y), 0.0)
    s = jnp.dot(s_ref[...], ws_ref[...] * sc_s,
                preferred_element_type=jnp.float32)
    s = jnp.maximum(s + (bs_ref[...] * sc_s + sh_s), 0.0)
    o_ref[...] = jnp.concatenate([c_ref[...], x, s], axis=1)


def _final_call(x, s, c, st, wp, bp, gp, bep, ws, bs, gs, bes):
    nblk = _BN // _RB_F
    co2 = 3 + 2 * _CO
    return pl.pallas_call(
        _final_body,
        grid=(nblk,),
        in_specs=[
            pl.BlockSpec((_RB_F, _CF), lambda i: (i, 0)),
            pl.BlockSpec((_RB_F, _CS), lambda i: (i, 0)),
            pl.BlockSpec((_RB_F, 3), lambda i: (i, 0)),
            pl.BlockSpec((8, _CO), lambda i: (0, 0)),
            pl.BlockSpec((_CF, _CO), lambda i: (0, 0)),
            pl.BlockSpec((1, _CO), lambda i: (0, 0)),
            pl.BlockSpec((1, _CO), lambda i: (0, 0)),
            pl.BlockSpec((1, _CO), lambda i: (0, 0)),
            pl.BlockSpec((_CS, _CO), lambda i: (0, 0)),
            pl.BlockSpec((1, _CO), lambda i: (0, 0)),
            pl.BlockSpec((1, _CO), lambda i: (0, 0)),
            pl.BlockSpec((1, _CO), lambda i: (0, 0)),
        ],
        out_specs=pl.BlockSpec((_RB_F, co2), lambda i: (i, 0)),
        out_shape=jax.ShapeDtypeStruct((_BN, co2), jnp.float32),
    )(x, s, c, st, wp, bp, gp, bep, ws, bs, gs, bes)


def kernel(curr_coords, curr_feat, skip_coords, skip_feat, upsampling_idxs,
           W_proj, b_proj, g_proj, be_proj, W_skip, b_skip, g_skip, be_skip):
    idx_t = jnp.transpose(upsampling_idxs.reshape(_BN, _K)).reshape(_K * _BN)
    table = curr_feat.reshape(_B * _M, _CF)
    inter = _gather_max(idx_t, table)

    skip2 = skip_feat.reshape(_BN, _CS)
    coords2 = skip_coords.reshape(_BN, 3)
    r = lambda v: v.reshape(1, _CO)
    stats = _stats_call(inter, skip2, W_proj, r(b_proj), W_skip, r(b_skip))
    out = _final_call(inter, skip2, coords2, stats, W_proj, r(b_proj),
                      r(g_proj), r(be_proj), W_skip, r(b_skip), r(g_skip),
                      r(be_skip))
    return out.reshape(_B, _N, 3 + 2 * _CO)


# speedup vs baseline: 17.1322x; 1.1987x over previous
import functools

import jax
import jax.numpy as jnp
from jax import lax
from jax.experimental import pallas as pl
from jax.experimental.pallas import tpu as pltpu
from jax.experimental.pallas import tpu_sc as plsc

_B, _M, _N, _K = 8, 4096, 16384, 3
_CF, _CS, _CO = 64, 32, 64
_BN = _B * _N
_EPS = 1e-5
_OUTW = 3 + 2 * _CO
_NEG = -3.0e38


_NW = 32
_PTS = _BN // _NW
_CH = 128
_NCH = _PTS // _CH


def _gather_max_body(idx_hbm, table_hbm, out_hbm, fidx_v, rows_v, out_v,
                     gsem0, gsem1, osem0, osem1):
    wid = lax.axis_index("s") * 2 + lax.axis_index("c")
    base = wid * _PTS
    row_off = (base // _N) * _M
    gsems = (gsem0, gsem1)
    osems = (osem0, osem1)

    def prefetch(c, buf):
        p0 = base + c * _CH
        for k in range(_K):
            pltpu.sync_copy(idx_hbm.at[pl.ds(k * _BN + p0, _CH)],
                            fidx_v.at[buf, k])
        for k in range(_K):
            for g in range(_CH // 16):
                sl = pl.ds(g * 16, 16)
                fidx_v[buf, k, sl] = fidx_v[buf, k, sl] + row_off
        for k in range(_K):
            pltpu.async_copy(table_hbm.at[fidx_v.at[buf, k]],
                             rows_v.at[buf, k], gsems[buf])

    def compute(c, buf):
        p0 = base + c * _CH
        for k in range(_K):
            pltpu.make_async_copy(table_hbm.at[fidx_v.at[buf, k]],
                                  rows_v.at[buf, k], gsems[buf]).wait()

        @pl.when(c >= 2)
        def _drain_out():
            pltpu.make_async_copy(out_v.at[buf],
                                  out_hbm.at[pl.ds(p0 - 2 * _CH, _CH)],
                                  osems[buf]).wait()

        @plsc.parallel_loop(0, _CH, unroll=2)
        def _max_body(p):
            for c4 in range(_CF // 16):
                sl = pl.ds(c4 * 16, 16)
                m = jnp.maximum(rows_v[buf, 0, p, sl], rows_v[buf, 1, p, sl])
                out_v[buf, p, sl] = jnp.maximum(m, rows_v[buf, 2, p, sl])

        pltpu.async_copy(out_v.at[buf], out_hbm.at[pl.ds(p0, _CH)],
                         osems[buf])

    prefetch(0, 0)
    prefetch(1, 1)

    def pair(jj, carry):
        c0 = jj * 2
        compute(c0, 0)

        @pl.when(c0 + 2 < _NCH)
        def _pf0():
            prefetch(c0 + 2, 0)

        compute(c0 + 1, 1)

        @pl.when(c0 + 3 < _NCH)
        def _pf1():
            prefetch(c0 + 3, 1)

        return carry

    lax.fori_loop(0, _NCH // 2, pair, 0)
    for buf, c in ((0, _NCH - 2), (1, _NCH - 1)):
        pltpu.make_async_copy(out_v.at[buf],
                              out_hbm.at[pl.ds(base + c * _CH, _CH)],
                              osems[buf]).wait()


@functools.cache
def _gather_max_kernel():
    return pl.kernel(
        _gather_max_body,
        out_type=jax.ShapeDtypeStruct((_BN, _CF), jnp.float32),
        mesh=plsc.VectorSubcoreMesh(core_axis_name="c", subcore_axis_name="s"),
        compiler_params=pltpu.CompilerParams(use_tc_tiling_on_sc=False),
        scratch_types=[
            pltpu.VMEM((2, _K, _CH), jnp.int32),
            pltpu.VMEM((2, _K, _CH, _CF), jnp.float32),
            pltpu.VMEM((2, _CH, _CF), jnp.float32),
            pltpu.SemaphoreType.DMA,
            pltpu.SemaphoreType.DMA,
            pltpu.SemaphoreType.DMA,
            pltpu.SemaphoreType.DMA,
        ],
    )


def _gather_max(idx_t, table):
    return _gather_max_kernel()(idx_t, table)


_RB_S = 2048


def _stats_body(x_ref, s_ref, wp_ref, bp_ref, gp_ref, bep_ref, ws_ref,
                bs_ref, gs_ref, bes_ref, wc_ref, wx_ref, wso_ref, br_ref,
                gx_acc, gs_acc, sx_acc, ss_acc):
    i = pl.program_id(0)
    nblk = pl.num_programs(0)

    @pl.when(i == 0)
    def _init():
        gx_acc[...] = jnp.zeros_like(gx_acc)
        gs_acc[...] = jnp.zeros_like(gs_acc)
        sx_acc[...] = jnp.zeros_like(sx_acc)
        ss_acc[...] = jnp.zeros_like(ss_acc)

    x = x_ref[...]
    s = s_ref[...]
    dn = (((0,), (0,)), ((), ()))
    gx_acc[...] += lax.dot_general(x, x, dn, preferred_element_type=jnp.float32)
    gs_acc[...] += lax.dot_general(s, s, dn, preferred_element_type=jnp.float32)
    ones = jnp.ones((8, _RB_S), jnp.float32)
    sx_acc[...] += jnp.dot(ones, x, preferred_element_type=jnp.float32)
    ss_acc[...] += jnp.dot(ones, s, preferred_element_type=jnp.float32)

    @pl.when(i == nblk - 1)
    def _fold():
        nb = jnp.float32(_BN)
        wp = wp_ref[...]
        bp = bp_ref[...]
        sxw = jnp.dot(sx_acc[0:1, :], wp,
                      preferred_element_type=jnp.float32) / nb
        mean_y = sxw + bp
        gw = jnp.dot(gx_acc[...], wp, preferred_element_type=jnp.float32)
        ey2 = (jnp.sum(wp * gw, 0, keepdims=True) / nb + 2.0 * bp * sxw
               + bp * bp)
        var_y = ey2 - mean_y * mean_y
        sc_y = gp_ref[...] * lax.rsqrt(var_y + _EPS)
        bx_eff = (bp - mean_y) * sc_y + bep_ref[...]
        wx_eff = wp * sc_y
        ws = ws_ref[...]
        bs = bs_ref[...]
        ssw = jnp.dot(ss_acc[0:1, 0:_CS], ws,
                      preferred_element_type=jnp.float32) / nb
        mean_s = ssw + bs
        gws = jnp.dot(gs_acc[...], ws, preferred_element_type=jnp.float32)
        es2 = (jnp.sum(ws * gws, 0, keepdims=True) / nb + 2.0 * bs * ssw
               + bs * bs)
        var_s = es2 - mean_s * mean_s
        sc_s = gs_ref[...] * lax.rsqrt(var_s + _EPS)
        bs_eff = (bs - mean_s) * sc_s + bes_ref[...]
        ws_eff = ws * sc_s

        wx_ref[...] = jnp.concatenate(
            [jnp.zeros((_CF, 3), jnp.float32), wx_eff,
             jnp.zeros((_CF, _CO), jnp.float32)], axis=1)
        wso_ref[...] = jnp.concatenate(
            [jnp.zeros((_CS, 3 + _CO), jnp.float32), ws_eff], axis=1)
        r_i = lax.broadcasted_iota(jnp.int32, (8, _OUTW), 0)
        c_i = lax.broadcasted_iota(jnp.int32, (8, _OUTW), 1)
        wc_ref[...] = jnp.where((r_i == c_i) & (r_i < 3), 1.0, 0.0)
        bias_row = jnp.concatenate(
            [jnp.zeros((1, 3), jnp.float32), bx_eff, bs_eff], axis=1)
        floor_row = jnp.where(c_i[0:1, :] < 3, _NEG, 0.0)
        br_ref[...] = jnp.concatenate(
            [bias_row, floor_row, jnp.zeros((6, _OUTW), jnp.float32)], axis=0)


def _stats_call(x, s, wp, bp, gp, bep, ws, bs, gs, bes):
    nblk = _BN // _RB_S
    full = lambda shp: pl.BlockSpec(shp, lambda i: (0, 0))
    return pl.pallas_call(
        _stats_body,
        grid=(nblk,),
        in_specs=[
            pl.BlockSpec((_RB_S, _CF), lambda i: (i, 0)),
            pl.BlockSpec((_RB_S, _CS), lambda i: (i, 0)),
            full((_CF, _CO)), full((1, _CO)), full((1, _CO)), full((1, _CO)),
            full((_CS, _CO)), full((1, _CO)), full((1, _CO)), full((1, _CO)),
        ],
        out_specs=[
            full((8, _OUTW)), full((_CF, _OUTW)), full((_CS, _OUTW)),
            full((8, _OUTW)),
        ],
        out_shape=[
            jax.ShapeDtypeStruct((8, _OUTW), jnp.float32),
            jax.ShapeDtypeStruct((_CF, _OUTW), jnp.float32),
            jax.ShapeDtypeStruct((_CS, _OUTW), jnp.float32),
            jax.ShapeDtypeStruct((8, _OUTW), jnp.float32),
        ],
        scratch_shapes=[
            pltpu.VMEM((_CF, _CF), jnp.float32),
            pltpu.VMEM((_CS, _CS), jnp.float32),
            pltpu.VMEM((8, _CF), jnp.float32),
            pltpu.VMEM((8, _CS), jnp.float32),
        ],
    )(x, s, wp, bp, gp, bep, ws, bs, gs, bes)


_RB_F = 2048


def _final_body(x_ref, s_ref, c_ref, wc_ref, wx_ref, wso_ref, br_ref, o_ref):
    acc = jnp.dot(c_ref[...], wc_ref[0:3, :],
                  preferred_element_type=jnp.float32)
    acc += jnp.dot(x_ref[...], wx_ref[...],
                   preferred_element_type=jnp.float32)
    acc += jnp.dot(s_ref[...], wso_ref[...],
                   preferred_element_type=jnp.float32)
    o_ref[...] = jnp.maximum(acc + br_ref[0:1, :], br_ref[1:2, :])


def _final_call(x, s, c, wc, wx, wso, br):
    nblk = _BN // _RB_F
    full = lambda shp: pl.BlockSpec(shp, lambda i: (0, 0))
    return pl.pallas_call(
        _final_body,
        grid=(nblk,),
        in_specs=[
            pl.BlockSpec((_RB_F, _CF), lambda i: (i, 0)),
            pl.BlockSpec((_RB_F, _CS), lambda i: (i, 0)),
            pl.BlockSpec((_RB_F, 3), lambda i: (i, 0)),
            full((8, _OUTW)), full((_CF, _OUTW)), full((_CS, _OUTW)),
            full((8, _OUTW)),
        ],
        out_specs=pl.BlockSpec((_RB_F, _OUTW), lambda i: (i, 0)),
        out_shape=jax.ShapeDtypeStruct((_BN, _OUTW), jnp.float32),
    )(x, s, c, wc, wx, wso, br)


def kernel(curr_coords, curr_feat, skip_coords, skip_feat, upsampling_idxs,
           W_proj, b_proj, g_proj, be_proj, W_skip, b_skip, g_skip, be_skip):
    idx_t = jnp.transpose(upsampling_idxs.reshape(_BN, _K)).reshape(_K * _BN)
    table = curr_feat.reshape(_B * _M, _CF)
    inter = _gather_max(idx_t, table)

    skip2 = skip_feat.reshape(_BN, _CS)
    coords2 = skip_coords.reshape(_BN, 3)
    r = lambda v: v.reshape(1, _CO)
    wc, wx, wso, br = _stats_call(inter, skip2, W_proj, r(b_proj), r(g_proj),
                                  r(be_proj), W_skip, r(b_skip), r(g_skip),
                                  r(be_skip))
    out = _final_call(inter, skip2, coords2, wc, wx, wso, br)
    return out.reshape(_B, _N, _OUTW)
